# Initial kernel scaffold; baseline (speedup 1.0000x reference)
import jax
import jax.numpy as jnp
from jax.experimental import pallas as pl


def kernel(x, edge_index, W_l, W_r, b):
    raise NotImplementedError("write your pallas kernel here")


# speedup vs baseline: 1.3699x; 1.3699x over previous
import functools

import jax
import jax.numpy as jnp
from jax import lax
from jax.experimental import pallas as pl
from jax.experimental.pallas import tpu as pltpu
from jax.experimental.pallas import tpu_sc as plsc

NC = 2
NS = 16
NW = NC * NS
CHUNK = 128
LANES = 16


def _sc_gather_scatter(src, dst, x, n_pad, n_chunks, rows_per_tile):
    D = x.shape[1]
    mesh = plsc.VectorSubcoreMesh(core_axis_name="c", subcore_axis_name="s",
                                  num_cores=NC, num_subcores=NS)

    full = rows_per_tile // CHUNK

    @functools.partial(
        pl.kernel,
        out_type=[
            jax.ShapeDtypeStruct((NC * n_pad * D,), jnp.float32),
            jax.ShapeDtypeStruct((NC * n_pad,), jnp.float32),
        ],
        mesh=mesh,
        scratch_types=[
            pltpu.VMEM((CHUNK,), jnp.int32),
            pltpu.VMEM((CHUNK,), jnp.int32),
            pltpu.VMEM((CHUNK,), jnp.int32),
            pltpu.VMEM((CHUNK,), jnp.int32),
            pltpu.VMEM((CHUNK, D), jnp.float32),
            pltpu.VMEM((CHUNK,), jnp.float32),
            pltpu.VMEM_SHARED((n_pad * D,), jnp.float32),
            pltpu.VMEM_SHARED((n_pad,), jnp.float32),
            pltpu.SemaphoreType.DMA,
        ],
    )
    def k(src_h, dst_h, x_h, iota_h, agg_h, cnt_h, sidx, didx, eidx, iota128,
          rows, ones1, agg_s, cnt_s, sem):
        c = lax.axis_index("c")
        s = lax.axis_index("s")
        wid = s * NC + c

        pltpu.sync_copy(iota_h.at[pl.ds(0, CHUNK)], iota128)

        zero16 = jnp.zeros((LANES,), jnp.float32)

        def zrow(i, carry):
            def zcol(j, carry2):
                rows[i, pl.ds(j * LANES, LANES)] = zero16
                return carry2
            lax.fori_loop(0, D // LANES, zcol, 0)
            return carry
        lax.fori_loop(0, CHUNK, zrow, 0)

        def z1(j, carry):
            ones1[pl.ds(j * LANES, LANES)] = zero16
            return carry
        lax.fori_loop(0, CHUNK // LANES, z1, 0)

        def fill_eidx(r):
            base = r * D

            def fill(j, c2):
                eidx[pl.ds(j * LANES, LANES)] = (
                    base + iota128[pl.ds(j * LANES, LANES)])
                return c2
            lax.fori_loop(0, CHUNK // LANES, fill, 0)

        r0 = s * rows_per_tile

        def zagg(i, carry):
            fill_eidx(r0 + i)
            pltpu.sync_copy(rows.at[0], agg_s.at[eidx])
            return carry
        lax.fori_loop(0, rows_per_tile, zagg, 0)

        def zcnt(i, carry):
            pltpu.sync_copy(iota_h.at[pl.ds(r0 + i * CHUNK, CHUNK)], didx)
            pltpu.sync_copy(ones1, cnt_s.at[didx])
            return carry
        lax.fori_loop(0, full, zcnt, 0)

        one16 = jnp.ones((LANES,), jnp.float32)

        def orow(j, carry):
            ones1[pl.ds(j * LANES, LANES)] = one16
            return carry
        lax.fori_loop(0, CHUNK // LANES, orow, 0)

        plsc.subcore_barrier()

        ebase = wid * (n_chunks * CHUNK)

        def body(j, carry):
            eb = ebase + j * CHUNK
            pltpu.sync_copy(src_h.at[pl.ds(eb, CHUNK)], sidx)
            pltpu.sync_copy(dst_h.at[pl.ds(eb, CHUNK)], didx)
            pltpu.async_copy(x_h.at[sidx], rows, sem).wait()
            pltpu.sync_copy(ones1, cnt_s.at[didx], add=True)

            def per_group(g, c2):
                dvec = didx[pl.ds(g * LANES, LANES)]
                for l in range(LANES):
                    fill_eidx(dvec[l])
                    pltpu.sync_copy(rows.at[g * LANES + l], agg_s.at[eidx],
                                    add=True)
                return c2
            lax.fori_loop(0, CHUNK // LANES, per_group, 0)
            return carry
        lax.fori_loop(0, n_chunks, body, 0)

        plsc.subcore_barrier()

        def wagg(i, carry):
            r = r0 + i
            fill_eidx(r)
            pltpu.sync_copy(agg_s.at[eidx], ones1)
            pltpu.sync_copy(ones1, agg_h.at[pl.ds((c * n_pad + r) * D, CHUNK)])
            return carry
        lax.fori_loop(0, rows_per_tile, wagg, 0)

        def wcnt(i, carry):
            r = r0 + i * CHUNK
            pltpu.sync_copy(iota_h.at[pl.ds(r, CHUNK)], didx)
            pltpu.sync_copy(cnt_s.at[didx], ones1)
            pltpu.sync_copy(ones1, cnt_h.at[pl.ds(c * n_pad + r, CHUNK)])
            return carry
        lax.fori_loop(0, full, wcnt, 0)

    iota = jnp.arange(n_pad, dtype=jnp.int32)
    return k(src, dst, x, iota)


def _tc_finish(aggp, cntp, x, wl_t, wr_t, b2, block_n):
    N, D = x.shape
    n_blocks = N // block_n

    def body(agg_ref, cnt_ref, x_ref, wl_ref, wr_ref, b_ref, o_ref):
        p = agg_ref[0] + agg_ref[1]
        cnt = cnt_ref[0] + cnt_ref[1]
        recip = 1.0 / jnp.maximum(cnt[:, :1], 1.0)
        mean = p * recip
        acc = jnp.dot(mean, wl_ref[...], preferred_element_type=jnp.float32)
        acc += jnp.dot(x_ref[...], wr_ref[...], preferred_element_type=jnp.float32)
        acc += b_ref[...]
        o_ref[...] = jnp.maximum(acc, 0.0)

    return pl.pallas_call(
        body,
        grid=(n_blocks,),
        in_specs=[
            pl.BlockSpec((NC, block_n, D), lambda i: (0, i, 0)),
            pl.BlockSpec((NC, block_n, 1), lambda i: (0, i, 0)),
            pl.BlockSpec((block_n, D), lambda i: (i, 0)),
            pl.BlockSpec((D, D), lambda i: (0, 0)),
            pl.BlockSpec((D, D), lambda i: (0, 0)),
            pl.BlockSpec((1, D), lambda i: (0, 0)),
        ],
        out_specs=pl.BlockSpec((block_n, D), lambda i: (i, 0)),
        out_shape=jax.ShapeDtypeStruct((N, D), jnp.float32),
    )(aggp, cntp, x, wl_t, wr_t, b2)


def kernel(x, edge_index, W_l, W_r, b):
    N, D = x.shape
    E = edge_index.shape[1]

    e_w = (-(-E // NW) + CHUNK - 1) // CHUNK * CHUNK
    n_chunks = e_w // CHUNK
    e_pad = e_w * NW

    n_pad = -(-(N + 1) // (NS * CHUNK)) * (NS * CHUNK)
    rows_per_tile = n_pad // NS

    src = jnp.concatenate([edge_index[0], jnp.zeros((e_pad - E,), jnp.int32)])
    dst = jnp.concatenate([edge_index[1], jnp.full((e_pad - E,), N, jnp.int32)])

    aggp, cntp = _sc_gather_scatter(src, dst, x, n_pad, n_chunks, rows_per_tile)
    aggp = aggp.reshape(NC, n_pad, D)
    cntp = cntp.reshape(NC, n_pad, 1)

    return _tc_finish(aggp, cntp, x, W_l.T, W_r.T, b.reshape(1, D),
                      block_n=1000)
